# baseline (speedup 1.0000x reference)
import functools

import numpy as np

import jax
import jax.numpy as jnp
from jax import lax
from jax.experimental import pallas as pl
from jax.experimental.pallas import tpu as pltpu
from jax.experimental.pallas import tpu_sc as plsc

N = 10000
NPAD = 10240
DIN = 128
DOUT = 128
WL = 144
NEG = 0.2
NC, NS, LANES = 2, 16, 16
NW = NC * NS
C = 48
UN = 4
ROWS_PER_TILE = NPAD // NS

_XR_PERM = np.empty((DIN,), np.int32)
for _k in range(DIN // 32):
    for _i in range(16):
        _XR_PERM[32 * _k + 2 * _i] = 32 * _k + _i
        _XR_PERM[32 * _k + 2 * _i + 1] = 32 * _k + 16 + _i


def _proj_body(x_ref, wl_ref, wr_ref, xlh_ref, xr_ref):
    xb = x_ref[...]
    xr_ref[...] = jnp.dot(
        xb, wr_ref[...], preferred_element_type=jnp.float32
    ).astype(jnp.bfloat16)
    xlh_ref[:, :DIN] = jnp.dot(xb, wl_ref[...], preferred_element_type=jnp.float32)
    tail = lax.broadcasted_iota(jnp.int32, (xb.shape[0], WL - DIN), 1)
    xlh_ref[:, DIN:] = jnp.where(tail == 0, 1.0, 0.0).astype(jnp.float32)


def _project(x_pad, W_l, W_rp):
    BP = 1024
    return pl.pallas_call(
        _proj_body,
        grid=(NPAD // BP,),
        in_specs=[
            pl.BlockSpec((BP, DIN), lambda i: (i, 0)),
            pl.BlockSpec((DIN, DOUT), lambda i: (0, 0)),
            pl.BlockSpec((DIN, DOUT), lambda i: (0, 0)),
        ],
        out_specs=[
            pl.BlockSpec((BP, WL), lambda i: (i, 0)),
            pl.BlockSpec((BP, DOUT), lambda i: (i, 0)),
        ],
        out_shape=[
            jax.ShapeDtypeStruct((NPAD, WL), jnp.float32),
            jax.ShapeDtypeStruct((NPAD, DOUT), jnp.bfloat16),
        ],
    )(x_pad, W_l, W_rp)


def _comb_body(acc_ref, b_ref, out_ref):
    a = acc_ref[0] + acc_ref[1]
    num = a[:, :DOUT]
    den = a[:, DOUT:DOUT + 1]
    out_ref[...] = num / (den + 1e-16) + b_ref[...]


def _combine(acc, b2):
    BC = 1000
    return pl.pallas_call(
        _comb_body,
        grid=(N // BC,),
        in_specs=[
            pl.BlockSpec((NC, BC, WL), lambda i: (0, i, 0)),
            pl.BlockSpec((1, DOUT), lambda i: (0, 0)),
        ],
        out_specs=pl.BlockSpec((BC, DOUT), lambda i: (i, 0)),
        out_shape=jax.ShapeDtypeStruct((N, DOUT), jnp.float32),
    )(acc, b2)


def _sc_body(nch, xlh, xrb, src_h, dst_h, zero_h, att_h, acc_out,
             si0, si1, di0, di1, di2, di3, rl0, rl1, rl2, rl3, rr0, rr1,
             att_v, scr, p_v, acc_s,
             sgl0, sgl1, sgr0, sgr1, ss0, ss1, sxs0, sxs1, sxd0, sxd1):
    c = lax.axis_index("c")
    s = lax.axis_index("s")
    wid = s * NC + c
    rbase = s * ROWS_PER_TILE
    ept = nch * C
    tb = wid * ept
    si = (si0, si1)
    di = (di0, di1, di2, di3)
    rl = (rl0, rl1, rl2, rl3)
    rr = (rr0, rr1)
    sgl, sgr, ss = (sgl0, sgl1), (sgr0, sgr1), (ss0, ss1)
    sxs, sxd = (sxs0, sxs1), (sxd0, sxd1)

    pltpu.sync_copy(src_h.at[pl.ds(tb, C)], si0)
    pltpu.sync_copy(dst_h.at[pl.ds(tb, C)], di0)
    pltpu.async_copy(xlh.at[si0], rl0, sgl0)
    pltpu.async_copy(xrb.at[di0], rr0, sgr0)
    pltpu.async_copy(src_h.at[pl.ds(tb + C, C)], si1, sxs1)
    pltpu.async_copy(dst_h.at[pl.ds(tb + C, C)], di1, sxd1)
    pltpu.sync_copy(zero_h.at[pl.ds(0, C)], rl3)
    pltpu.sync_copy(zero_h.at[pl.ds(rbase, ROWS_PER_TILE)],
                    acc_s.at[pl.ds(rbase, ROWS_PER_TILE)])
    pltpu.sync_copy(att_h, att_v)
    plsc.subcore_barrier()
    pltpu.async_copy(rl3, acc_s.at[di0], ss0, add=True)
    pltpu.async_copy(rl3, acc_s.at[di0], ss1, add=True)

    att8 = [att_v[pl.ds(k * 16, 16)] for k in range(DIN // 16)]
    lane = lax.iota(jnp.int32, 16)
    e0 = jnp.where(lane == 0, 1.0, 0.0).astype(jnp.float32)

    def splat(v, idx):
        return v.at[idx].get(mode="promise_in_bounds")

    SG = 2

    def compute(rl_b, rr_b):
        def group(g, carry2):
            sb = g * SG
            t = jnp.zeros((16,), jnp.float32)
            zls = []
            for e in range(SG):
                row = sb + e
                zl = []
                accs = []
                for k in range(DIN // 32):
                    pair = rr_b[row, pl.ds(k * 32, 32)]
                    lo, hi = plsc.unpack(pair,
                                         format=plsc.PackFormat.INTERLEAVED)
                    x0 = rl_b[row, pl.ds((2 * k) * 16, 16)]
                    x1 = rl_b[row, pl.ds((2 * k + 1) * 16, 16)]
                    zl += [x0, x1]
                    z0 = x0 + lo
                    z1 = x1 + hi
                    w0 = jnp.maximum(z0, NEG * z0)
                    w1 = jnp.maximum(z1, NEG * z1)
                    accs.append(w0 * att8[2 * k] + w1 * att8[2 * k + 1])
                acc = (accs[0] + accs[1]) + (accs[2] + accs[3])
                for sh in (8, 4, 2, 1):
                    acc = acc + splat(acc, lane ^ sh)
                t = jnp.where(lane == e, acc, t)
                zls.append(zl)
            p16 = jnp.exp(t)
            for e in range(SG):
                row = sb + e
                pe = splat(p16, jnp.full((16,), e, jnp.int32))
                for kk in range(DIN // 16):
                    rl_b[row, pl.ds(kk * 16, 16)] = zls[e][kk] * pe
                rl_b[row, pl.ds(DIN, 16)] = e0 * pe
            return carry2
        lax.fori_loop(0, C // SG, group, 0)

    def quad(q, carry):
        for u in range(UN):
            j = UN * q + u
            b = u % 2
            pltpu.make_async_copy(xlh.at[si[b]], rl[u], sgl[b]).wait()
            pltpu.make_async_copy(xrb.at[di[u]], rr[b], sgr[b]).wait()
            pltpu.make_async_copy(rl[(u + 2) % UN], acc_s.at[di[(u + 2) % UN]],
                                  ss[b]).wait()
            pltpu.make_async_copy(src_h.at[pl.ds(tb, C)], si[1 - b],
                                  sxs[1 - b]).wait()
            pltpu.make_async_copy(dst_h.at[pl.ds(tb, C)], di[(u + 1) % UN],
                                  sxd[1 - b]).wait()
            pltpu.async_copy(xlh.at[si[1 - b]], rl[(u + 1) % UN], sgl[1 - b])
            pltpu.async_copy(xrb.at[di[(u + 1) % UN]], rr[1 - b], sgr[1 - b])
            base2 = tb + jnp.minimum(j + 2, nch - 1) * C
            pltpu.async_copy(src_h.at[pl.ds(base2, C)], si[b], sxs[b])
            pltpu.async_copy(dst_h.at[pl.ds(base2, C)], di[(u + 2) % UN],
                             sxd[b])
            compute(rl[u], rr[b])
            pltpu.async_copy(rl[u], acc_s.at[di[u]], ss[b], add=True)
        return carry

    lax.fori_loop(0, nch // UN, quad, 0)
    pltpu.make_async_copy(rl0, acc_s.at[di0], ss0).wait()
    pltpu.make_async_copy(rl0, acc_s.at[di0], ss1).wait()
    pltpu.make_async_copy(xlh.at[si0], rl0, sgl0).wait()
    pltpu.make_async_copy(xrb.at[di0], rr0, sgr0).wait()
    pltpu.make_async_copy(src_h.at[pl.ds(tb, C)], si1, sxs1).wait()
    pltpu.make_async_copy(dst_h.at[pl.ds(tb, C)], di3, sxd1).wait()
    plsc.subcore_barrier()
    pltpu.sync_copy(acc_s.at[pl.ds(rbase, ROWS_PER_TILE)],
                    acc_out.at[c, pl.ds(rbase, ROWS_PER_TILE)])


@functools.cache
def _make_sc(nch):
    mesh = plsc.VectorSubcoreMesh(core_axis_name="c", subcore_axis_name="s")
    return pl.kernel(
        functools.partial(_sc_body, nch),
        out_type=jax.ShapeDtypeStruct((NC, NPAD, WL), jnp.float32),
        mesh=mesh,
        compiler_params=pltpu.CompilerParams(needs_layout_passes=False,
                                             use_tc_tiling_on_sc=False),
        scratch_types=[
            pltpu.VMEM((C,), jnp.int32),
            pltpu.VMEM((C,), jnp.int32),
            pltpu.VMEM((C,), jnp.int32),
            pltpu.VMEM((C,), jnp.int32),
            pltpu.VMEM((C,), jnp.int32),
            pltpu.VMEM((C,), jnp.int32),
            pltpu.VMEM((C, WL), jnp.float32),
            pltpu.VMEM((C, WL), jnp.float32),
            pltpu.VMEM((C, WL), jnp.float32),
            pltpu.VMEM((C, WL), jnp.float32),
            pltpu.VMEM((C, DOUT), jnp.bfloat16),
            pltpu.VMEM((C, DOUT), jnp.bfloat16),
            pltpu.VMEM((DIN,), jnp.float32),
            pltpu.VMEM((16, 16), jnp.float32),
            pltpu.VMEM((16,), jnp.float32),
            pltpu.VMEM_SHARED((NPAD, WL), jnp.float32),
            pltpu.SemaphoreType.DMA,
            pltpu.SemaphoreType.DMA,
            pltpu.SemaphoreType.DMA,
            pltpu.SemaphoreType.DMA,
            pltpu.SemaphoreType.DMA,
            pltpu.SemaphoreType.DMA,
            pltpu.SemaphoreType.DMA,
            pltpu.SemaphoreType.DMA,
            pltpu.SemaphoreType.DMA,
            pltpu.SemaphoreType.DMA,
        ],
    )


def kernel(x, edge_index, W_l, W_r, att, b):
    E = edge_index.shape[1]
    src = edge_index[0].astype(jnp.int32)
    dst = edge_index[1].astype(jnp.int32)
    loop = jnp.arange(N, dtype=jnp.int32)
    e_tot = E + N
    ept = -(-e_tot // (NW * UN * C)) * UN * C
    pad = NW * ept - e_tot
    dummy = jnp.full((pad,), N, jnp.int32)
    src_all = jnp.concatenate([src, loop, dummy])
    dst_all = jnp.concatenate([dst, loop, dummy])
    x_pad = jnp.pad(x, ((0, NPAD - N), (0, 0)))
    W_rp = W_r[:, _XR_PERM]
    xlh, xrb = _project(x_pad, W_l, W_rp)
    zero_h = jnp.zeros((NPAD, WL), jnp.float32)
    acc = _make_sc(int(ept // C))(xlh, xrb, src_all, dst_all, zero_h,
                                  att.reshape(DIN))
    return _combine(acc, b.reshape(1, DOUT))


# speedup vs baseline: 1.0741x; 1.0741x over previous
import functools

import numpy as np

import jax
import jax.numpy as jnp
from jax import lax
from jax.experimental import pallas as pl
from jax.experimental.pallas import tpu as pltpu
from jax.experimental.pallas import tpu_sc as plsc

N = 10000
NPAD = 10240
DIN = 128
DOUT = 128
WL = 144
NEG = 0.2
NC, NS, LANES = 2, 16, 16
NW = NC * NS
C = 48
UN = 4
ROWS_PER_TILE = NPAD // NS

_XR_PERM = np.empty((DIN,), np.int32)
for _k in range(DIN // 32):
    for _i in range(16):
        _XR_PERM[32 * _k + 2 * _i] = 32 * _k + _i
        _XR_PERM[32 * _k + 2 * _i + 1] = 32 * _k + 16 + _i


def _proj_body(x_ref, wl_ref, wr_ref, xlh_ref, xr_ref):
    xb = x_ref[...]
    xr_ref[...] = jnp.dot(
        xb, wr_ref[...], preferred_element_type=jnp.float32
    ).astype(jnp.bfloat16)
    xlh_ref[:, :DIN] = jnp.dot(xb, wl_ref[...], preferred_element_type=jnp.float32)
    tail = lax.broadcasted_iota(jnp.int32, (xb.shape[0], WL - DIN), 1)
    xlh_ref[:, DIN:] = jnp.where(tail == 0, 1.0, 0.0).astype(jnp.float32)


def _project(x_pad, W_l, W_rp):
    BP = 1024
    return pl.pallas_call(
        _proj_body,
        grid=(NPAD // BP,),
        in_specs=[
            pl.BlockSpec((BP, DIN), lambda i: (i, 0)),
            pl.BlockSpec((DIN, DOUT), lambda i: (0, 0)),
            pl.BlockSpec((DIN, DOUT), lambda i: (0, 0)),
        ],
        out_specs=[
            pl.BlockSpec((BP, WL), lambda i: (i, 0)),
            pl.BlockSpec((BP, DOUT), lambda i: (i, 0)),
        ],
        out_shape=[
            jax.ShapeDtypeStruct((NPAD, WL), jnp.float32),
            jax.ShapeDtypeStruct((NPAD, DOUT), jnp.bfloat16),
        ],
    )(x_pad, W_l, W_rp)


def _comb_body(acc_ref, b_ref, out_ref):
    a = acc_ref[0] + acc_ref[1]
    num = a[:, :DOUT]
    den = a[:, DOUT:DOUT + 1]
    out_ref[...] = num / (den + 1e-16) + b_ref[...]


def _combine(acc, b2):
    BC = 1000
    return pl.pallas_call(
        _comb_body,
        grid=(N // BC,),
        in_specs=[
            pl.BlockSpec((NC, BC, WL), lambda i: (0, i, 0)),
            pl.BlockSpec((1, DOUT), lambda i: (0, 0)),
        ],
        out_specs=pl.BlockSpec((BC, DOUT), lambda i: (i, 0)),
        out_shape=jax.ShapeDtypeStruct((N, DOUT), jnp.float32),
    )(acc, b2)


def _sc_body(nch, xlh, xrb, src_h, dst_h, zero_h, att_h, acc_out,
             si0, si1, di0, di1, di2, di3, rl0, rl1, rl2, rl3, rr0, rr1,
             att_v, scr, p_v, acc_s,
             sgl0, sgl1, sgr0, sgr1, ss0, ss1, sxs0, sxs1, sxd0, sxd1):
    c = lax.axis_index("c")
    s = lax.axis_index("s")
    wid = s * NC + c
    rbase = s * ROWS_PER_TILE
    ept = nch * C
    tb = wid * ept
    si = (si0, si1)
    di = (di0, di1, di2, di3)
    rl = (rl0, rl1, rl2, rl3)
    rr = (rr0, rr1)
    sgl, sgr, ss = (sgl0, sgl1), (sgr0, sgr1), (ss0, ss1)
    sxs, sxd = (sxs0, sxs1), (sxd0, sxd1)

    pltpu.sync_copy(src_h.at[pl.ds(tb, C)], si0)
    pltpu.sync_copy(dst_h.at[pl.ds(tb, C)], di0)
    pltpu.async_copy(xlh.at[si0], rl0, sgl0)
    pltpu.async_copy(xrb.at[di0], rr0, sgr0)
    pltpu.async_copy(src_h.at[pl.ds(tb + C, C)], si1, sxs1)
    pltpu.async_copy(dst_h.at[pl.ds(tb + C, C)], di1, sxd1)
    pltpu.sync_copy(zero_h.at[pl.ds(0, C)], rl3)
    pltpu.sync_copy(zero_h.at[pl.ds(rbase, ROWS_PER_TILE)],
                    acc_s.at[pl.ds(rbase, ROWS_PER_TILE)])
    pltpu.sync_copy(att_h, att_v)
    plsc.subcore_barrier()
    pltpu.async_copy(rl3, acc_s.at[di0], ss0, add=True)
    pltpu.async_copy(rl3, acc_s.at[di0], ss1, add=True)

    att8 = [att_v[pl.ds(k * 16, 16)] for k in range(DIN // 16)]
    lane = lax.iota(jnp.int32, 16)
    e0 = jnp.where(lane == 0, 1.0, 0.0).astype(jnp.float32)

    def splat(v, idx):
        return v.at[idx].get(mode="promise_in_bounds")

    SG = 16

    def compute(rl_b, rr_b):
        def group(g, carry2):
            sb = g * SG
            t = jnp.zeros((16,), jnp.float32)
            for e in range(SG):
                row = sb + e
                accs = []
                for k in range(DIN // 32):
                    pair = rr_b[row, pl.ds(k * 32, 32)]
                    lo, hi = plsc.unpack(pair,
                                         format=plsc.PackFormat.INTERLEAVED)
                    z0 = rl_b[row, pl.ds((2 * k) * 16, 16)] + lo
                    z1 = rl_b[row, pl.ds((2 * k + 1) * 16, 16)] + hi
                    w0 = jnp.maximum(z0, NEG * z0)
                    w1 = jnp.maximum(z1, NEG * z1)
                    accs.append(w0 * att8[2 * k] + w1 * att8[2 * k + 1])
                acc = (accs[0] + accs[1]) + (accs[2] + accs[3])
                for sh in (8, 4, 2, 1):
                    acc = acc + splat(acc, lane ^ sh)
                t = jnp.where(lane == e, acc, t)
            p16 = jnp.exp(t)
            for e in range(SG):
                row = sb + e
                pe = splat(p16, jnp.full((16,), e, jnp.int32))
                for kk in range(DIN // 16):
                    rl_b[row, pl.ds(kk * 16, 16)] = \
                        rl_b[row, pl.ds(kk * 16, 16)] * pe
                rl_b[row, pl.ds(DIN, 16)] = e0 * pe
            return carry2
        lax.fori_loop(0, C // SG, group, 0)

    def quad(q, carry):
        for u in range(UN):
            j = UN * q + u
            b = u % 2
            pltpu.make_async_copy(xlh.at[si[b]], rl[u], sgl[b]).wait()
            pltpu.make_async_copy(xrb.at[di[u]], rr[b], sgr[b]).wait()
            pltpu.make_async_copy(rl[(u + 2) % UN], acc_s.at[di[(u + 2) % UN]],
                                  ss[b]).wait()
            pltpu.make_async_copy(src_h.at[pl.ds(tb, C)], si[1 - b],
                                  sxs[1 - b]).wait()
            pltpu.make_async_copy(dst_h.at[pl.ds(tb, C)], di[(u + 1) % UN],
                                  sxd[1 - b]).wait()
            pltpu.async_copy(xlh.at[si[1 - b]], rl[(u + 1) % UN], sgl[1 - b])
            pltpu.async_copy(xrb.at[di[(u + 1) % UN]], rr[1 - b], sgr[1 - b])
            base2 = tb + jnp.minimum(j + 2, nch - 1) * C
            pltpu.async_copy(src_h.at[pl.ds(base2, C)], si[b], sxs[b])
            pltpu.async_copy(dst_h.at[pl.ds(base2, C)], di[(u + 2) % UN],
                             sxd[b])
            compute(rl[u], rr[b])
            pltpu.async_copy(rl[u], acc_s.at[di[u]], ss[b], add=True)
        return carry

    lax.fori_loop(0, nch // UN, quad, 0)
    pltpu.make_async_copy(rl0, acc_s.at[di0], ss0).wait()
    pltpu.make_async_copy(rl0, acc_s.at[di0], ss1).wait()
    pltpu.make_async_copy(xlh.at[si0], rl0, sgl0).wait()
    pltpu.make_async_copy(xrb.at[di0], rr0, sgr0).wait()
    pltpu.make_async_copy(src_h.at[pl.ds(tb, C)], si1, sxs1).wait()
    pltpu.make_async_copy(dst_h.at[pl.ds(tb, C)], di3, sxd1).wait()
    plsc.subcore_barrier()
    pltpu.sync_copy(acc_s.at[pl.ds(rbase, ROWS_PER_TILE)],
                    acc_out.at[c, pl.ds(rbase, ROWS_PER_TILE)])


@functools.cache
def _make_sc(nch):
    mesh = plsc.VectorSubcoreMesh(core_axis_name="c", subcore_axis_name="s")
    return pl.kernel(
        functools.partial(_sc_body, nch),
        out_type=jax.ShapeDtypeStruct((NC, NPAD, WL), jnp.float32),
        mesh=mesh,
        compiler_params=pltpu.CompilerParams(needs_layout_passes=False,
                                             use_tc_tiling_on_sc=False),
        scratch_types=[
            pltpu.VMEM((C,), jnp.int32),
            pltpu.VMEM((C,), jnp.int32),
            pltpu.VMEM((C,), jnp.int32),
            pltpu.VMEM((C,), jnp.int32),
            pltpu.VMEM((C,), jnp.int32),
            pltpu.VMEM((C,), jnp.int32),
            pltpu.VMEM((C, WL), jnp.float32),
            pltpu.VMEM((C, WL), jnp.float32),
            pltpu.VMEM((C, WL), jnp.float32),
            pltpu.VMEM((C, WL), jnp.float32),
            pltpu.VMEM((C, DOUT), jnp.bfloat16),
            pltpu.VMEM((C, DOUT), jnp.bfloat16),
            pltpu.VMEM((DIN,), jnp.float32),
            pltpu.VMEM((16, 16), jnp.float32),
            pltpu.VMEM((16,), jnp.float32),
            pltpu.VMEM_SHARED((NPAD, WL), jnp.float32),
            pltpu.SemaphoreType.DMA,
            pltpu.SemaphoreType.DMA,
            pltpu.SemaphoreType.DMA,
            pltpu.SemaphoreType.DMA,
            pltpu.SemaphoreType.DMA,
            pltpu.SemaphoreType.DMA,
            pltpu.SemaphoreType.DMA,
            pltpu.SemaphoreType.DMA,
            pltpu.SemaphoreType.DMA,
            pltpu.SemaphoreType.DMA,
        ],
    )


def kernel(x, edge_index, W_l, W_r, att, b):
    E = edge_index.shape[1]
    src = edge_index[0].astype(jnp.int32)
    dst = edge_index[1].astype(jnp.int32)
    loop = jnp.arange(N, dtype=jnp.int32)
    e_tot = E + N
    ept = -(-e_tot // (NW * UN * C)) * UN * C
    pad = NW * ept - e_tot
    dummy = jnp.full((pad,), N, jnp.int32)
    src_all = jnp.concatenate([src, loop, dummy])
    dst_all = jnp.concatenate([dst, loop, dummy])
    x_pad = jnp.pad(x, ((0, NPAD - N), (0, 0)))
    W_rp = W_r[:, _XR_PERM]
    xlh, xrb = _project(x_pad, W_l, W_rp)
    zero_h = jnp.zeros((NPAD, WL), jnp.float32)
    acc = _make_sc(int(ept // C))(xlh, xrb, src_all, dst_all, zero_h,
                                  att.reshape(DIN))
    return _combine(acc, b.reshape(1, DOUT))
